# Initial kernel scaffold; baseline (speedup 1.0000x reference)
import jax
import jax.numpy as jnp
from jax.experimental import pallas as pl


def kernel(x, W0, W1, W2, gamma0, gamma1, gamma2, beta0, beta1, beta2):
    raise NotImplementedError("write your pallas kernel here")


# speedup vs baseline: 3.2951x; 3.2951x over previous
import functools

import jax
import jax.numpy as jnp
from jax import lax
from jax.experimental import pallas as pl
from jax.experimental.pallas import tpu as pltpu

_B, _C, _N, _O = 8, 64, 1024, 128
_KMAX = 40
_NB = 256


def _topk_body(xf_ref, xc_ref, idx_ref, s_scr):
    xf = xf_ref[0]
    xc = xc_ref[0]
    dn = (((0,), (0,)), ((), ()))
    t = lax.dot_general(xc, xf, dn, preferred_element_type=jnp.float32)
    ones = jnp.ones((_C, 1), jnp.float32)
    xxn = lax.dot_general(xc * xc, ones, dn,
                          preferred_element_type=jnp.float32)
    xxm = jnp.sum(xf * xf, axis=0, keepdims=True)
    s_scr[...] = (2.0 * t - xxn) - xxm
    iota_m = lax.broadcasted_iota(jnp.int32, (_NB, _N), 1)
    eye = (lax.broadcasted_iota(jnp.int32, (_NB, _NB), 0) ==
           lax.broadcasted_iota(jnp.int32, (_NB, _NB), 1)).astype(jnp.float32)

    def body(j, _):
        s = s_scr[...]
        m = jnp.max(s, axis=1, keepdims=True)
        am = jnp.min(jnp.where(s == m, iota_m, _N * 2), axis=1, keepdims=True)
        amr = lax.dot_general(am.astype(jnp.float32), eye, dn,
                              precision=lax.Precision.HIGHEST,
                              preferred_element_type=jnp.float32)
        idx_ref[0, pl.ds(j, 1), :] = amr.astype(jnp.int32)
        s_scr[...] = jnp.where(iota_m == am, -jnp.inf, s)
        return 0

    lax.fori_loop(0, _KMAX, body, 0, unroll=False)


def _uc_body(w_ref, x_ref, u_ref, c_ref):
    w = w_ref[0]
    x2 = x_ref[...]
    a = w[:, :_C]
    bm = w[:, _C:] - w[:, :_C]
    u_ref[0] = jnp.dot(a, x2, preferred_element_type=jnp.float32)
    c_ref[0] = jnp.dot(bm, x2, preferred_element_type=jnp.float32)


def _gather_body(idx_ref, u_ref, c_ref, mx_ref, st_ref, mx_scr, s1_scr, sq_scr):
    i = pl.program_id(0)
    k_dyn = lax.shift_left(10, i)
    u = u_ref[0]
    c = c_ref[0]
    iota_m = lax.broadcasted_iota(jnp.int32, (_N, _N), 0)
    mx_scr[...] = jnp.full((_O, _N), -jnp.inf, jnp.float32)
    s1_scr[...] = jnp.zeros((_O, _N), jnp.float32)
    sq_scr[...] = jnp.zeros((_O, _N), jnp.float32)
    dn = (((1,), (0,)), ((), ()))

    def body(j, _):
        row = idx_ref[0, pl.ds(j, 1), :]
        e = (iota_m == row).astype(jnp.float32)
        r = lax.dot_general(u, e, dn, preferred_element_type=jnp.float32)
        mx_scr[...] = jnp.maximum(mx_scr[...], r)
        s1_scr[...] = s1_scr[...] + r
        sq_scr[...] = sq_scr[...] + r * r
        return 0

    lax.fori_loop(0, k_dyn, body, 0, unroll=False)

    mx_ref[0, 0] = mx_scr[...]
    s1 = s1_scr[...]
    sq = sq_scr[...]
    st_ref[0, 0, :, 0:1] = jnp.sum(sq, axis=1, keepdims=True)
    st_ref[0, 0, :, 1:2] = jnp.sum(s1, axis=1, keepdims=True)
    st_ref[0, 0, :, 2:3] = jnp.sum(s1 * c, axis=1, keepdims=True)
    st_ref[0, 0, :, 3:4] = jnp.sum(c, axis=1, keepdims=True)
    st_ref[0, 0, :, 4:5] = jnp.sum(c * c, axis=1, keepdims=True)
    st_ref[0, 0, :, 5:8] = jnp.zeros((_O, 3), jnp.float32)


def _final_body(mx_ref, c_ref, mean_ref, scl_ref, off_ref, out_ref):
    y = mx_ref[0, 0] + c_ref[0]
    z = (y - mean_ref[0]) * scl_ref[0] + off_ref[0]
    out_ref[0, 0] = jnp.where(z > 0, z, 0.2 * z)


@jax.jit
def kernel(x, W0, W1, W2, gamma0, gamma1, gamma2, beta0, beta1, beta2):
    f32 = jnp.float32
    x = x.astype(f32)

    idx = pl.pallas_call(
        _topk_body,
        grid=(_B, _N // _NB),
        in_specs=[
            pl.BlockSpec((1, _C, _N), lambda b, nb: (b, 0, 0)),
            pl.BlockSpec((1, _C, _NB), lambda b, nb: (b, 0, nb)),
        ],
        out_specs=pl.BlockSpec((1, _KMAX, _NB), lambda b, nb: (b, 0, nb)),
        out_shape=jax.ShapeDtypeStruct((_B, _KMAX, _N), jnp.int32),
        scratch_shapes=[pltpu.VMEM((_NB, _N), f32)],
    )(x, x)

    x2d = jnp.transpose(x, (1, 0, 2)).reshape(_C, _B * _N)
    Ws = jnp.stack([W0, W1, W2]).astype(f32)
    u3, c3 = pl.pallas_call(
        _uc_body,
        grid=(3,),
        in_specs=[
            pl.BlockSpec((1, _O, 2 * _C), lambda i: (i, 0, 0)),
            pl.BlockSpec((_C, _B * _N), lambda i: (0, 0)),
        ],
        out_specs=[
            pl.BlockSpec((1, _O, _B * _N), lambda i: (i, 0, 0)),
            pl.BlockSpec((1, _O, _B * _N), lambda i: (i, 0, 0)),
        ],
        out_shape=[
            jax.ShapeDtypeStruct((3, _O, _B * _N), f32),
            jax.ShapeDtypeStruct((3, _O, _B * _N), f32),
        ],
    )(Ws, x2d)

    mx, st = pl.pallas_call(
        _gather_body,
        grid=(3, _B),
        in_specs=[
            pl.BlockSpec((1, _KMAX, _N), lambda i, b: (b, 0, 0)),
            pl.BlockSpec((1, _O, _N), lambda i, b: (i, 0, b)),
            pl.BlockSpec((1, _O, _N), lambda i, b: (i, 0, b)),
        ],
        out_specs=[
            pl.BlockSpec((1, 1, _O, _N), lambda i, b: (i, b, 0, 0)),
            pl.BlockSpec((1, 1, _O, 8), lambda i, b: (i, b, 0, 0)),
        ],
        out_shape=[
            jax.ShapeDtypeStruct((3, _B, _O, _N), f32),
            jax.ShapeDtypeStruct((3, _B, _O, 8), f32),
        ],
        scratch_shapes=[
            pltpu.VMEM((_O, _N), f32),
            pltpu.VMEM((_O, _N), f32),
            pltpu.VMEM((_O, _N), f32),
        ],
    )(idx, u3, c3)

    ks = jnp.array([10.0, 20.0, 40.0], f32)[:, None]
    sb = jnp.sum(st, axis=1)
    cnt = ks * (_B * _N)
    s1_tot = sb[:, :, 1] + ks * sb[:, :, 3]
    s2_tot = sb[:, :, 0] + 2.0 * sb[:, :, 2] + ks * sb[:, :, 4]
    mean = s1_tot / cnt
    var = s2_tot / cnt - mean * mean
    gam = jnp.stack([gamma0, gamma1, gamma2]).astype(f32)
    bet = jnp.stack([beta0, beta1, beta2]).astype(f32)
    scl = gam * lax.rsqrt(var + 1e-5)
    mean3 = mean[:, :, None]
    scl3 = scl[:, :, None]
    off3 = bet[:, :, None]

    out = pl.pallas_call(
        _final_body,
        grid=(3, _B),
        in_specs=[
            pl.BlockSpec((1, 1, _O, _N), lambda i, b: (i, b, 0, 0)),
            pl.BlockSpec((1, _O, _N), lambda i, b: (i, 0, b)),
            pl.BlockSpec((1, _O, 1), lambda i, b: (i, 0, 0)),
            pl.BlockSpec((1, _O, 1), lambda i, b: (i, 0, 0)),
            pl.BlockSpec((1, _O, 1), lambda i, b: (i, 0, 0)),
        ],
        out_specs=pl.BlockSpec((1, 1, _O, _N), lambda i, b: (i, b, 0, 0)),
        out_shape=jax.ShapeDtypeStruct((3, _B, _O, _N), f32),
    )(mx, c3, mean3, scl3, off3)

    return (out[0], out[1], out[2])
